# baseline (speedup 1.0000x reference)
import functools

import jax
import jax.numpy as jnp
from jax import lax
from jax.experimental import pallas as pl
from jax.experimental.pallas import tpu as pltpu
from jax.experimental.pallas import tpu_sc as plsc

N = 50000
E = 800000
B = 4
ISZ = 66
OUT = 64
NM = 5
W = 16
NBLK = 18
FP = NBLK * W

NC = 2
NS = 16
BPC = NBLK // NC
C = 1440
NCH = 35
GROUPS = C // 16
NP = 50048
SPS = NP // NS
EP = NS * C * NCH
EPSP = EP // NS
NP8 = NP // 8
SP8 = SPS // 8
TCOLS = 9
TN = 544


@functools.partial(
    pl.kernel,
    out_type=[jax.ShapeDtypeStruct((NBLK, NP, W), jnp.float32),
              jax.ShapeDtypeStruct((NBLK, NP, W), jnp.float32),
              jax.ShapeDtypeStruct((TCOLS, NP, 128), jnp.float32)],
    mesh=plsc.VectorSubcoreMesh(core_axis_name="c", subcore_axis_name="s"),
    compiler_params=pltpu.CompilerParams(
        use_tc_tiling_on_sc=False, needs_layout_passes=False),
    scratch_types=[
        pltpu.VMEM((C,), jnp.int32),
        pltpu.VMEM((C,), jnp.int32),
        pltpu.VMEM((C,), jnp.int32),
        pltpu.VMEM((C,), jnp.int32),
        pltpu.VMEM((C,), jnp.float32),
        pltpu.VMEM((C,), jnp.float32),
        pltpu.VMEM((C, W), jnp.float32),
        pltpu.VMEM((C, W), jnp.float32),
        pltpu.VMEM((C, W), jnp.float32),
        pltpu.SemaphoreType.DMA,
        pltpu.SemaphoreType.DMA,
        pltpu.SemaphoreType.DMA,
        pltpu.SemaphoreType.DMA,
        pltpu.VMEM_SHARED((NP, W), jnp.float32),
    ],
)
def _sc_chain(x0b, r0, c0, v0, r1, c1, v1, zrow,
              y1, y3, x4d,
              colsv0, colsv1, idxv0, idxv1, valsv0, valsv1, G0, G1, S,
              gsem0, gsem1, isem0, isem1, accum):
    cid = lax.axis_index("c")
    sid = lax.axis_index("s")
    i16 = lax.iota(jnp.int32, 16)

    bufs = ((colsv0, idxv0, valsv0, G0, gsem0, isem0),
            (colsv1, idxv1, valsv1, G1, gsem1, isem1))

    def run_pass(src, rows, cols, vals, dst, j, m):
        pltpu.sync_copy(zrow, accum.at[pl.ds(sid * SPS, SPS)])
        plsc.subcore_barrier()

        def issue_idx(k, b):
            cb, ib, vb, _, _, isem = bufs[b]
            off = sid * EPSP + k * C
            pltpu.async_copy(cols.at[pl.ds(off, C)], cb, isem)
            pltpu.async_copy(rows.at[pl.ds(off, C)], ib, isem)
            pltpu.async_copy(vals.at[pl.ds(off, C)], vb, isem)

        def wait_idx(b):
            cb, ib, vb, _, _, isem = bufs[b]
            pltpu.make_async_copy(cols.at[pl.ds(0, C)], cb, isem).wait()
            pltpu.make_async_copy(rows.at[pl.ds(0, C)], ib, isem).wait()
            pltpu.make_async_copy(vals.at[pl.ds(0, C)], vb, isem).wait()

        def issue_gather(b):
            cb, _, _, Gb, gsem, _ = bufs[b]
            pltpu.async_copy(src.at[j].at[cb], Gb, gsem)

        def wait_gather(b):
            cb, _, _, Gb, gsem, _ = bufs[b]
            pltpu.make_async_copy(src.at[j].at[cb], Gb, gsem).wait()

        def scale_scatter(b):
            cb, ib, vb, Gb, _, _ = bufs[b]

            @plsc.parallel_loop(0, GROUPS)
            def scale_body(t):
                v = vb[pl.ds(t * 16, 16)]
                base = t * 16
                for e in range(16):
                    sv = jnp.take_along_axis(
                        v, jnp.full((16,), e, jnp.int32), axis=0)
                    S[base + e, :] = Gb[base + e, :] * sv
            pltpu.sync_copy(S, accum.at[ib], add=True)

        issue_idx(0, 0)
        wait_idx(0)
        issue_gather(0)
        issue_idx(1, 1)
        wait_gather(0)
        wait_idx(1)
        issue_gather(1)
        scale_scatter(0)
        issue_idx(2, 0)

        def pair_body(g, _):
            k = 2 * g + 1
            wait_gather(1)
            wait_idx(0)
            issue_gather(0)
            scale_scatter(1)
            issue_idx(k + 2, 1)
            wait_gather(0)
            wait_idx(1)
            issue_gather(1)
            scale_scatter(0)
            issue_idx(k + 3, 0)
            return 0

        lax.fori_loop(0, (NCH - 3) // 2, pair_body, 0, unroll=False)
        wait_gather(1)
        wait_idx(0)
        issue_gather(0)
        scale_scatter(1)
        wait_gather(0)
        scale_scatter(0)
        plsc.subcore_barrier()
        if dst is not None:
            pltpu.sync_copy(accum.at[pl.ds(sid * SPS, SPS)],
                            dst.at[j].at[pl.ds(sid * SPS, SPS)])
        cc = (m - 1) * FP + j * W
        t0 = cc // 128
        o0 = lax.rem(cc, 128)
        pltpu.sync_copy(
            accum.at[pl.ds(sid * SPS, SPS)],
            x4d.at[t0].at[pl.ds(sid * SPS, SPS), pl.ds(o0, W)])
        plsc.subcore_barrier()

    def block_body(jl, _):
        j = cid * BPC + jl
        run_pass(x0b, r0, c0, v0, y1, j, 1)
        run_pass(y1, r0, c0, v0, None, j, 2)
        run_pass(y1, r1, c1, v1, y3, j, 3)
        run_pass(y3, r1, c1, v1, None, j, 4)
        return 0

    lax.fori_loop(0, BPC, block_body, 0, unroll=False)


def _mm_kernel(xr_ref, x4_ref, w0_ref, wt_ref, b_ref, o_ref):
    acc = jnp.dot(x4_ref[0], wt_ref[0], preferred_element_type=jnp.float32)
    for t in range(1, TCOLS):
        acc += jnp.dot(x4_ref[t], wt_ref[t],
                       preferred_element_type=jnp.float32)
    for b in range(B):
        o_ref[b] = (
            acc[:, b * OUT:(b + 1) * OUT]
            + jnp.dot(xr_ref[b], w0_ref[...],
                      preferred_element_type=jnp.float32)
            + b_ref[...]
        )


def _matmul(xr, x4d, w0p, wt, bias_row):
    return pl.pallas_call(
        _mm_kernel,
        grid=(pl.cdiv(N, TN),),
        in_specs=[
            pl.BlockSpec((B, TN, ISZ), lambda i: (0, i, 0)),
            pl.BlockSpec((TCOLS, TN, 128), lambda i: (0, i, 0)),
            pl.BlockSpec((ISZ, OUT), lambda i: (0, 0)),
            pl.BlockSpec((TCOLS, 128, B * OUT), lambda i: (0, 0, 0)),
            pl.BlockSpec((1, OUT), lambda i: (0, 0)),
        ],
        out_specs=pl.BlockSpec((B, TN, OUT), lambda i: (0, i, 0)),
        out_shape=jax.ShapeDtypeStruct((B, N, OUT), jnp.float32),
    )(xr, x4d, w0p, wt, bias_row)


def kernel(inputs, weight, biases, s0_rows, s0_cols, s0_vals, s1_rows, s1_cols, s1_vals):
    w = weight.reshape(ISZ, NM, OUT)
    w0, w1, w2, w3, w4 = (w[:, m] for m in range(NM))
    wm = jnp.stack([w0 - w2, w1 - w4, 2.0 * w2, w3, 2.0 * w4], axis=0)
    wm = jnp.pad(wm, ((0, 0), (0, FP // B - ISZ), (0, 0)))
    eye = jnp.eye(B, dtype=jnp.float32)
    wbig = wm[:, :, None, None, :] * eye[None, None, :, :, None]
    wbig = wbig.reshape(NM * FP, B * OUT)

    xr = inputs.reshape(B, N, ISZ)
    x0 = jnp.transpose(xr, (1, 2, 0)).reshape(N, ISZ * B)
    x0f = jnp.pad(x0, ((0, NP - N), (0, FP - ISZ * B)))
    x0b = x0f.reshape(NP, NBLK, W).transpose(1, 0, 2)

    zrow = jnp.zeros((SPS, W), jnp.float32)
    epad = ((0, EP - E),)
    r0, c0, v0, r1, c1, v1 = (jnp.pad(a, epad) for a in (
        s0_rows, s0_cols, s0_vals, s1_rows, s1_cols, s1_vals))
    _, _, x4d = _sc_chain(x0b, r0, c0, v0, r1, c1, v1, zrow)

    w0p = wm[0, :ISZ]
    wt = wbig[FP:].reshape(TCOLS, 128, B * OUT)
    out3 = _matmul(xr, x4d, w0p, wt, biases[None, :])
    return out3.reshape(B, N * OUT)


# speedup vs baseline: 1.0037x; 1.0037x over previous
import functools

import jax
import jax.numpy as jnp
from jax import lax
from jax.experimental import pallas as pl
from jax.experimental.pallas import tpu as pltpu
from jax.experimental.pallas import tpu_sc as plsc

N = 50000
E = 800000
B = 4
ISZ = 66
OUT = 64
NM = 5
W = 16
NBLK = 18
FP = NBLK * W

NC = 2
NS = 16
BPC = NBLK // NC
C = 1120
NCH = 45
GROUPS = C // 16
NP = 50048
SPS = NP // NS
EP = NS * C * NCH
EPSP = EP // NS
NP8 = NP // 8
SP8 = SPS // 8
TCOLS = 9
TN = 544


@functools.partial(
    pl.kernel,
    out_type=[jax.ShapeDtypeStruct((NBLK, NP, W), jnp.float32),
              jax.ShapeDtypeStruct((NBLK, NP, W), jnp.float32),
              jax.ShapeDtypeStruct((TCOLS, NP, 128), jnp.float32)],
    mesh=plsc.VectorSubcoreMesh(core_axis_name="c", subcore_axis_name="s"),
    compiler_params=pltpu.CompilerParams(
        use_tc_tiling_on_sc=False, needs_layout_passes=False),
    scratch_types=[
        pltpu.VMEM((C,), jnp.int32),
        pltpu.VMEM((C,), jnp.int32),
        pltpu.VMEM((C,), jnp.int32),
        pltpu.VMEM((C,), jnp.int32),
        pltpu.VMEM((C,), jnp.float32),
        pltpu.VMEM((C,), jnp.float32),
        pltpu.VMEM((C, W), jnp.float32),
        pltpu.VMEM((C, W), jnp.float32),
        pltpu.VMEM((C, W), jnp.float32),
        pltpu.VMEM((C, W), jnp.float32),
        pltpu.SemaphoreType.DMA,
        pltpu.SemaphoreType.DMA,
        pltpu.SemaphoreType.DMA,
        pltpu.SemaphoreType.DMA,
        pltpu.SemaphoreType.DMA,
        pltpu.SemaphoreType.DMA,
        pltpu.SemaphoreType.DMA,
        pltpu.SemaphoreType.DMA,
        pltpu.VMEM_SHARED((NP, W), jnp.float32),
    ],
)
def _sc_chain(x0b, r0, c0, v0, r1, c1, v1, zrow,
              y1, y3, x4d,
              colsv0, colsv1, idxv0, idxv1, valsv0, valsv1,
              G0, G1, S0, S1,
              gsem0, gsem1, isem0, isem1, rsem0, rsem1, ssem0, ssem1,
              accum):
    cid = lax.axis_index("c")
    sid = lax.axis_index("s")
    i16 = lax.iota(jnp.int32, 16)

    bufs = ((colsv0, idxv0, valsv0, G0, S0, gsem0, isem0, rsem0, ssem0),
            (colsv1, idxv1, valsv1, G1, S1, gsem1, isem1, rsem1, ssem1))

    def run_pass(src, rows, cols, vals, dst, j, m):
        pltpu.sync_copy(zrow, accum.at[pl.ds(sid * SPS, SPS)])
        plsc.subcore_barrier()

        def issue_idx(k, b):
            cb, _, vb, _, _, _, isem, _, _ = bufs[b]
            off = sid * EPSP + k * C
            pltpu.async_copy(cols.at[pl.ds(off, C)], cb, isem)
            pltpu.async_copy(vals.at[pl.ds(off, C)], vb, isem)

        def wait_idx(b):
            cb, _, vb, _, _, _, isem, _, _ = bufs[b]
            pltpu.make_async_copy(cols.at[pl.ds(0, C)], cb, isem).wait()
            pltpu.make_async_copy(vals.at[pl.ds(0, C)], vb, isem).wait()

        def issue_rows(k, b):
            _, ib, _, _, _, _, _, rsem, _ = bufs[b]
            off = sid * EPSP + k * C
            pltpu.async_copy(rows.at[pl.ds(off, C)], ib, rsem)

        def wait_rows(b):
            _, ib, _, _, _, _, _, rsem, _ = bufs[b]
            pltpu.make_async_copy(rows.at[pl.ds(0, C)], ib, rsem).wait()

        def issue_gather(b):
            cb, _, _, Gb, _, gsem, _, _, _ = bufs[b]
            pltpu.async_copy(src.at[j].at[cb], Gb, gsem)

        def wait_gather(b):
            cb, _, _, Gb, _, gsem, _, _, _ = bufs[b]
            pltpu.make_async_copy(src.at[j].at[cb], Gb, gsem).wait()

        def issue_scatter(b):
            _, ib, _, _, Sb, _, _, _, ssem = bufs[b]
            pltpu.async_copy(Sb, accum.at[ib], ssem, add=True)

        def wait_scatter(b):
            _, ib, _, _, Sb, _, _, _, ssem = bufs[b]
            pltpu.make_async_copy(Sb, accum.at[ib], ssem).wait()

        def scale(b):
            _, _, vb, Gb, Sb, _, _, _, _ = bufs[b]

            @plsc.parallel_loop(0, GROUPS)
            def scale_body(t):
                v = vb[pl.ds(t * 16, 16)]
                base = t * 16
                for e in range(16):
                    sv = jnp.take_along_axis(
                        v, jnp.full((16,), e, jnp.int32), axis=0)
                    Sb[base + e, :] = Gb[base + e, :] * sv

        def step(k, cur, w_scat, nxt1, nxt2):
            nxt = 1 - cur
            if w_scat:
                wait_scatter(cur)
            issue_rows(k, cur)
            wait_gather(cur)
            if nxt1:
                wait_idx(nxt)
                issue_gather(nxt)
            scale(cur)
            wait_rows(cur)
            issue_scatter(cur)
            if nxt2:
                issue_idx(k + 2, cur)

        issue_idx(0, 0)
        issue_idx(1, 1)
        wait_idx(0)
        issue_gather(0)
        step(0, 0, False, True, True)
        step(1, 1, False, True, True)

        def pair_body(g, _):
            k = 2 * g + 2
            step(k, 0, True, True, True)
            step(k + 1, 1, True, True, True)
            return 0

        lax.fori_loop(0, (NCH - 5) // 2, pair_body, 0, unroll=False)
        step(NCH - 3, 0, True, True, True)
        step(NCH - 2, 1, True, True, False)
        step(NCH - 1, 0, True, False, False)
        wait_scatter(1)
        wait_scatter(0)
        plsc.subcore_barrier()
        if dst is not None:
            pltpu.sync_copy(accum.at[pl.ds(sid * SPS, SPS)],
                            dst.at[j].at[pl.ds(sid * SPS, SPS)])
        cc = (m - 1) * FP + j * W
        t0 = cc // 128
        o0 = lax.rem(cc, 128)
        pltpu.sync_copy(
            accum.at[pl.ds(sid * SPS, SPS)],
            x4d.at[t0].at[pl.ds(sid * SPS, SPS), pl.ds(o0, W)])
        plsc.subcore_barrier()

    def block_body(jl, _):
        j = cid * BPC + jl
        run_pass(x0b, r0, c0, v0, y1, j, 1)
        run_pass(y1, r0, c0, v0, None, j, 2)
        run_pass(y1, r1, c1, v1, y3, j, 3)
        run_pass(y3, r1, c1, v1, None, j, 4)
        return 0

    lax.fori_loop(0, BPC, block_body, 0, unroll=False)


def _mm_kernel(xr_ref, x4_ref, w0_ref, wt_ref, b_ref, o_ref):
    acc = jnp.dot(x4_ref[0], wt_ref[0], preferred_element_type=jnp.float32)
    for t in range(1, TCOLS):
        acc += jnp.dot(x4_ref[t], wt_ref[t],
                       preferred_element_type=jnp.float32)
    for b in range(B):
        o_ref[b] = (
            acc[:, b * OUT:(b + 1) * OUT]
            + jnp.dot(xr_ref[b], w0_ref[...],
                      preferred_element_type=jnp.float32)
            + b_ref[...]
        )


def _matmul(xr, x4d, w0p, wt, bias_row):
    return pl.pallas_call(
        _mm_kernel,
        grid=(pl.cdiv(N, TN),),
        in_specs=[
            pl.BlockSpec((B, TN, ISZ), lambda i: (0, i, 0)),
            pl.BlockSpec((TCOLS, TN, 128), lambda i: (0, i, 0)),
            pl.BlockSpec((ISZ, OUT), lambda i: (0, 0)),
            pl.BlockSpec((TCOLS, 128, B * OUT), lambda i: (0, 0, 0)),
            pl.BlockSpec((1, OUT), lambda i: (0, 0)),
        ],
        out_specs=pl.BlockSpec((B, TN, OUT), lambda i: (0, i, 0)),
        out_shape=jax.ShapeDtypeStruct((B, N, OUT), jnp.float32),
    )(xr, x4d, w0p, wt, bias_row)


def kernel(inputs, weight, biases, s0_rows, s0_cols, s0_vals, s1_rows, s1_cols, s1_vals):
    w = weight.reshape(ISZ, NM, OUT)
    w0, w1, w2, w3, w4 = (w[:, m] for m in range(NM))
    wm = jnp.stack([w0 - w2, w1 - w4, 2.0 * w2, w3, 2.0 * w4], axis=0)
    wm = jnp.pad(wm, ((0, 0), (0, FP // B - ISZ), (0, 0)))
    eye = jnp.eye(B, dtype=jnp.float32)
    wbig = wm[:, :, None, None, :] * eye[None, None, :, :, None]
    wbig = wbig.reshape(NM * FP, B * OUT)

    xr = inputs.reshape(B, N, ISZ)
    x0 = jnp.transpose(xr, (1, 2, 0)).reshape(N, ISZ * B)
    x0f = jnp.pad(x0, ((0, NP - N), (0, FP - ISZ * B)))
    x0b = x0f.reshape(NP, NBLK, W).transpose(1, 0, 2)

    zrow = jnp.zeros((SPS, W), jnp.float32)
    epad = ((0, EP - E),)
    r0, c0, v0, r1, c1, v1 = (jnp.pad(a, epad) for a in (
        s0_rows, s0_cols, s0_vals, s1_rows, s1_cols, s1_vals))
    _, _, x4d = _sc_chain(x0b, r0, c0, v0, r1, c1, v1, zrow)

    w0p = wm[0, :ISZ]
    wt = wbig[FP:].reshape(TCOLS, 128, B * OUT)
    out3 = _matmul(xr, x4d, w0p, wt, biases[None, :])
    return out3.reshape(B, N * OUT)
